# Initial kernel scaffold; baseline (speedup 1.0000x reference)
import jax
import jax.numpy as jnp
from jax.experimental import pallas as pl


def kernel(edge_index, node_features, lap_pe, in_W, in_b, pe_W, pe_b, conv1_W, conv1_b, bn1_g, bn1_b, conv2_W, conv2_b, bn2_g, bn2_b, ff_W, ff_b, p1_W, p1_b, p2_W, p2_b, l1_W, l1_b, l2_W, l2_b):
    raise NotImplementedError("write your pallas kernel here")


# speedup vs baseline: 3.0085x; 3.0085x over previous
import functools

import jax
import jax.numpy as jnp
from jax import lax
from jax.experimental import pallas as pl
from jax.experimental.pallas import tpu as pltpu
from jax.experimental.pallas import tpu_sc as plsc

N = 10000
NPAD = 10240
E = 160000
H = 256
HH = 128
NSUB = 16
EPT = E // NSUB
CH = 128
NCH = NPAD // CH
ROWS_PT = NPAD // NSUB

_MESH = plsc.VectorSubcoreMesh(core_axis_name="c", subcore_axis_name="s")
_f32 = jnp.float32


@functools.partial(
    pl.kernel,
    out_type=jax.ShapeDtypeStruct((2 * NPAD,), _f32),
    mesh=_MESH,
    scratch_types=[
        pltpu.VMEM((NCH, CH), jnp.int32),
        pltpu.VMEM((CH,), _f32),
        pltpu.VMEM((ROWS_PT,), _f32),
        pltpu.VMEM_SHARED((NPAD,), _f32),
    ],
)
def _deg_kernel(idx_hbm, deg_hbm, idx_v, ones_v, zero_v, acc_sh):
    c = lax.axis_index("c")
    s = lax.axis_index("s")

    @pl.loop(0, CH // 16)
    def _(i):
        ones_v[pl.ds(i * 16, 16)] = jnp.ones((16,), _f32)

    @pl.loop(0, ROWS_PT // 16)
    def _(i):
        zero_v[pl.ds(i * 16, 16)] = jnp.zeros((16,), _f32)

    pltpu.sync_copy(zero_v, acc_sh.at[pl.ds(s * ROWS_PT, ROWS_PT)])
    pltpu.sync_copy(idx_hbm.at[c * NSUB + s], idx_v)
    plsc.subcore_barrier()

    @pl.loop(0, NCH)
    def _(k):
        pltpu.sync_copy(ones_v, acc_sh.at[idx_v.at[k]], add=True)

    plsc.subcore_barrier()
    pltpu.sync_copy(acc_sh.at[pl.ds(s * ROWS_PT, ROWS_PT)],
                    deg_hbm.at[pl.ds(c * NPAD + s * ROWS_PT, ROWS_PT)])


@functools.partial(
    pl.kernel,
    out_type=jax.ShapeDtypeStruct((2 * NPAD, HH), _f32),
    mesh=_MESH,
    scratch_types=[
        pltpu.VMEM((NCH, CH), jnp.int32),
        pltpu.VMEM((NCH, CH), jnp.int32),
        pltpu.VMEM((CH, HH), _f32),
        pltpu.VMEM_SHARED((NPAD, HH), _f32),
    ],
)
def _mp_kernel(sidx_hbm, didx_hbm, g_hbm, a_hbm, sidx, didx, rows, acc):
    c = lax.axis_index("c")
    s = lax.axis_index("s")

    @pl.loop(0, CH)
    def _(i):
        @pl.loop(0, HH // 16)
        def _(j):
            rows[i, pl.ds(j * 16, 16)] = jnp.zeros((16,), _f32)

    @pl.loop(0, ROWS_PT // CH)
    def _(k):
        pltpu.sync_copy(rows, acc.at[pl.ds(s * ROWS_PT + k * CH, CH)])

    pltpu.sync_copy(sidx_hbm.at[c * NSUB + s], sidx)
    pltpu.sync_copy(didx_hbm.at[s], didx)
    plsc.subcore_barrier()

    @pl.loop(0, NCH)
    def _(k):
        pltpu.sync_copy(g_hbm.at[sidx.at[k]], rows)
        pltpu.sync_copy(rows, acc.at[didx.at[k]], add=True)

    plsc.subcore_barrier()
    pltpu.sync_copy(acc.at[pl.ds(s * ROWS_PT, ROWS_PT)],
                    a_hbm.at[pl.ds(c * NPAD + s * ROWS_PT, ROWS_PT)])


def _write_g(g, g_o):
    pad = jnp.zeros((NPAD - N, HH), _f32)
    g_o[0:N, :] = g[:, 0:HH]
    g_o[N:NPAD, :] = pad
    g_o[NPAD:NPAD + N, :] = g[:, HH:H]
    g_o[NPAD + N:2 * NPAD, :] = pad


def _init_body(nf, pe, inW, inb, peW, peb, dout, din,
               h_o, g_o, nin_o, nout_o):
    do = dout[0:N, :]
    di = din[0:N, :]
    nout = jnp.where(do > 0, 1.0 / jnp.sqrt(do), 0.0)
    nin = jnp.where(di > 0, 1.0 / jnp.sqrt(di), 0.0)
    h = (jnp.dot(nf[...], inW[...], preferred_element_type=_f32) + inb[...]
         + jnp.dot(pe[...], peW[...], preferred_element_type=_f32) + peb[...])
    h_o[...] = h
    nin_o[...] = nin
    nout_o[...] = nout
    _write_g(h * nout, g_o)


_init_call = pl.pallas_call(
    _init_body,
    out_shape=(
        jax.ShapeDtypeStruct((N, H), _f32),
        jax.ShapeDtypeStruct((2 * NPAD, HH), _f32),
        jax.ShapeDtypeStruct((N, 1), _f32),
        jax.ShapeDtypeStruct((N, 1), _f32),
    ),
)


def _gconv_post(a, nin, W, b, bng, bnb):
    af = jnp.concatenate([a[0:N, :], a[NPAD:NPAD + N, :]], axis=1) * nin
    t = jnp.dot(af, W[...], preferred_element_type=_f32) + b
    mu = jnp.mean(t, axis=0, keepdims=True)
    d = t - mu
    var = jnp.mean(d * d, axis=0, keepdims=True)
    return jnp.maximum(d * (1.0 / jnp.sqrt(var + 1e-5)) * bng + bnb, 0.0)


def _mid_body(a, nin, nout, W, b, bng, bnb, g_o):
    x = _gconv_post(a[...], nin[...], W[...], b[...], bng[...], bnb[...])
    _write_g(x * nout[...], g_o)


_mid_call = pl.pallas_call(
    _mid_body,
    out_shape=jax.ShapeDtypeStruct((2 * NPAD, HH), _f32),
)


def _x2_body(a, nin, W, b, bng, bnb, x_o):
    x_o[...] = _gconv_post(a[...], nin[...], W[...], b[...], bng[...],
                           bnb[...])


_x2_call = pl.pallas_call(
    _x2_body,
    out_shape=jax.ShapeDtypeStruct((N, H), _f32),
)


def _ff_body(x, ffW, ffb, hin, nout, h_o, g_o):
    h = jnp.dot(x[...], ffW[...], preferred_element_type=_f32) + ffb[...] + hin[...]
    h_o[...] = h
    _write_g(h * nout[...], g_o)


_ff_call = pl.pallas_call(
    _ff_body,
    out_shape=(
        jax.ShapeDtypeStruct((N, H), _f32),
        jax.ShapeDtypeStruct((2 * NPAD, HH), _f32),
    ),
)


def _head_body(h, p1W, p1b, p2W, p2b, l1W, l1b, l2W, l2b, pose_o, label_o):
    hh = h[...]
    z = jnp.maximum(jnp.dot(hh, p1W[...], preferred_element_type=_f32) + p1b[...], 0.0)
    pose_o[...] = jnp.dot(z, p2W[...], preferred_element_type=_f32) + p2b[...]
    y = jnp.mean(hh, axis=0, keepdims=True)
    u = jnp.maximum(jnp.dot(y, l1W[...], preferred_element_type=_f32) + l1b[...], 0.0)
    label_o[...] = jnp.dot(u, l2W[...], preferred_element_type=_f32) + l2b[...]


_head_call = pl.pallas_call(
    _head_body,
    out_shape=(
        jax.ShapeDtypeStruct((N, 3), _f32),
        jax.ShapeDtypeStruct((1, 60), _f32),
    ),
)


def kernel(edge_index, node_features, lap_pe, in_W, in_b, pe_W, pe_b,
           conv1_W, conv1_b, bn1_g, bn1_b, conv2_W, conv2_b, bn2_g, bn2_b,
           ff_W, ff_b, p1_W, p1_b, p2_W, p2_b, l1_W, l1_b, l2_W, l2_b):
    src = edge_index[0].reshape(NSUB, EPT)
    dst = edge_index[1].reshape(NSUB, EPT)
    padc = jnp.full((NSUB, NPAD - EPT), N, jnp.int32)
    src3 = jnp.concatenate([src, padc], axis=1).reshape(NSUB, NCH, CH)
    dst3 = jnp.concatenate([dst, padc], axis=1).reshape(NSUB, NCH, CH)
    degidx = jnp.concatenate([src3, dst3], axis=0)
    sidx2 = jnp.concatenate([src3, src3 + NPAD], axis=0)

    deg = _deg_kernel(degidx)
    dout = deg[0:NPAD].reshape(NPAD, 1)
    din = deg[NPAD:2 * NPAD].reshape(NPAD, 1)

    r = lambda v: v.reshape(1, -1)
    h, g, nin, nout = _init_call(
        node_features, lap_pe, in_W, r(in_b), pe_W, r(pe_b), dout, din)

    for i in range(conv1_W.shape[0]):
        a = _mp_kernel(sidx2, dst3, g)
        g = _mid_call(a, nin, nout, conv1_W[i], r(conv1_b[i]),
                      r(bn1_g[i]), r(bn1_b[i]))
        a = _mp_kernel(sidx2, dst3, g)
        x = _x2_call(a, nin, conv2_W[i], r(conv2_b[i]),
                     r(bn2_g[i]), r(bn2_b[i]))
        h, g = _ff_call(x, ff_W[i], r(ff_b[i]), h, nout)

    pose, label = _head_call(h, p1_W, r(p1_b), p2_W, r(p2_b),
                             l1_W, r(l1_b), l2_W, r(l2_b))
    return (pose, label)


# speedup vs baseline: 3.4562x; 1.1488x over previous
import functools

import jax
import jax.numpy as jnp
from jax import lax
from jax.experimental import pallas as pl
from jax.experimental.pallas import tpu as pltpu
from jax.experimental.pallas import tpu_sc as plsc

N = 10000
NPAD = 10240
E = 160000
H = 256
HH = 128
NSUB = 16
EPT = E // NSUB
CH = 128
NCH = NPAD // CH
ROWS_PT = NPAD // NSUB

_MESH = plsc.VectorSubcoreMesh(core_axis_name="c", subcore_axis_name="s")
_f32 = jnp.float32


@functools.partial(
    pl.kernel,
    out_type=jax.ShapeDtypeStruct((2 * NPAD,), _f32),
    mesh=_MESH,
    scratch_types=[
        pltpu.VMEM((NCH, CH), jnp.int32),
        pltpu.VMEM((CH,), _f32),
        pltpu.VMEM((ROWS_PT,), _f32),
        pltpu.VMEM_SHARED((NPAD,), _f32),
    ],
)
def _deg_kernel(idx_hbm, deg_hbm, idx_v, ones_v, zero_v, acc_sh):
    c = lax.axis_index("c")
    s = lax.axis_index("s")

    @pl.loop(0, CH // 16)
    def _(i):
        ones_v[pl.ds(i * 16, 16)] = jnp.ones((16,), _f32)

    @pl.loop(0, ROWS_PT // 16)
    def _(i):
        zero_v[pl.ds(i * 16, 16)] = jnp.zeros((16,), _f32)

    pltpu.sync_copy(zero_v, acc_sh.at[pl.ds(s * ROWS_PT, ROWS_PT)])
    pltpu.sync_copy(idx_hbm.at[c * NSUB + s], idx_v)
    plsc.subcore_barrier()

    @pl.loop(0, NCH)
    def _(k):
        pltpu.sync_copy(ones_v, acc_sh.at[idx_v.at[k]], add=True)

    plsc.subcore_barrier()
    pltpu.sync_copy(acc_sh.at[pl.ds(s * ROWS_PT, ROWS_PT)],
                    deg_hbm.at[pl.ds(c * NPAD + s * ROWS_PT, ROWS_PT)])


@functools.partial(
    pl.kernel,
    out_type=jax.ShapeDtypeStruct((2 * NPAD, HH), _f32),
    mesh=_MESH,
    scratch_types=[
        pltpu.VMEM((NCH, CH), jnp.int32),
        pltpu.VMEM((NCH // 2, CH), jnp.int32),
        pltpu.VMEM((2, CH, HH), _f32),
        pltpu.VMEM_SHARED((NPAD, HH), _f32),
        pltpu.SemaphoreType.DMA,
        pltpu.SemaphoreType.DMA,
    ],
)
def _mp_kernel(sidx_hbm, didx_hbm, g_hbm, a_hbm, sidx, didx, rows, acc,
               sem0, sem1):
    c = lax.axis_index("c")
    s = lax.axis_index("s")

    @pl.loop(0, CH)
    def _(i):
        @pl.loop(0, HH // 16)
        def _(j):
            rows[0, i, pl.ds(j * 16, 16)] = jnp.zeros((16,), _f32)

    @pl.loop(0, ROWS_PT // CH)
    def _(k):
        pltpu.sync_copy(rows.at[0], acc.at[pl.ds(s * ROWS_PT + k * CH, CH)])

    pltpu.sync_copy(sidx_hbm.at[c * NSUB + s], sidx)
    plsc.subcore_barrier()

    sems = (sem0, sem1)
    HCH = NCH // 2

    def _gather_start(k, b, sem):
        pltpu.async_copy(g_hbm.at[sidx.at[k]], rows.at[b], sem)

    def _gather_wait(k, b, sem):
        pltpu.make_async_copy(g_hbm.at[sidx.at[k]], rows.at[b], sem).wait()

    _gather_start(0, 0, sem0)

    for p in range(2):
        pltpu.sync_copy(didx_hbm.at[s * 2 + p], didx)

        @pl.loop(p * HCH, (p + 1) * HCH, step=2)
        def _(k):
            for b in range(2):
                kk = k + b
                _gather_wait(kk, b, sems[b])

                @pl.when(kk + 1 < NCH)
                def _():
                    _gather_start(kk + 1, 1 - b, sems[1 - b])

                pltpu.sync_copy(rows.at[b], acc.at[didx.at[kk - p * HCH]],
                                add=True)

    plsc.subcore_barrier()
    pltpu.sync_copy(acc.at[pl.ds(s * ROWS_PT, ROWS_PT)],
                    a_hbm.at[pl.ds(c * NPAD + s * ROWS_PT, ROWS_PT)])


def _write_g(g, g_o):
    pad = jnp.zeros((NPAD - N, HH), _f32)
    g_o[0:N, :] = g[:, 0:HH]
    g_o[N:NPAD, :] = pad
    g_o[NPAD:NPAD + N, :] = g[:, HH:H]
    g_o[NPAD + N:2 * NPAD, :] = pad


def _init_body(nf, pe, inW, inb, peW, peb, dout, din,
               h_o, g_o, nin_o, nout_o):
    do = dout[0:N, :]
    di = din[0:N, :]
    nout = jnp.where(do > 0, 1.0 / jnp.sqrt(do), 0.0)
    nin = jnp.where(di > 0, 1.0 / jnp.sqrt(di), 0.0)
    h = (jnp.dot(nf[...], inW[...], preferred_element_type=_f32) + inb[...]
         + jnp.dot(pe[...], peW[...], preferred_element_type=_f32) + peb[...])
    h_o[...] = h
    nin_o[...] = nin
    nout_o[...] = nout
    _write_g(h * nout, g_o)


_init_call = pl.pallas_call(
    _init_body,
    out_shape=(
        jax.ShapeDtypeStruct((N, H), _f32),
        jax.ShapeDtypeStruct((2 * NPAD, HH), _f32),
        jax.ShapeDtypeStruct((N, 1), _f32),
        jax.ShapeDtypeStruct((N, 1), _f32),
    ),
)


def _gconv_post(a, nin, W, b, bng, bnb):
    af = jnp.concatenate([a[0:N, :], a[NPAD:NPAD + N, :]], axis=1) * nin
    t = jnp.dot(af, W[...], preferred_element_type=_f32) + b
    mu = jnp.mean(t, axis=0, keepdims=True)
    d = t - mu
    var = jnp.mean(d * d, axis=0, keepdims=True)
    return jnp.maximum(d * (1.0 / jnp.sqrt(var + 1e-5)) * bng + bnb, 0.0)


def _mid_body(a, nin, nout, W, b, bng, bnb, g_o):
    x = _gconv_post(a[...], nin[...], W[...], b[...], bng[...], bnb[...])
    _write_g(x * nout[...], g_o)


_mid_call = pl.pallas_call(
    _mid_body,
    out_shape=jax.ShapeDtypeStruct((2 * NPAD, HH), _f32),
)


def _x2_body(a, nin, W, b, bng, bnb, x_o):
    x_o[...] = _gconv_post(a[...], nin[...], W[...], b[...], bng[...],
                           bnb[...])


_x2_call = pl.pallas_call(
    _x2_body,
    out_shape=jax.ShapeDtypeStruct((N, H), _f32),
)


def _ff_body(x, ffW, ffb, hin, nout, h_o, g_o):
    h = jnp.dot(x[...], ffW[...], preferred_element_type=_f32) + ffb[...] + hin[...]
    h_o[...] = h
    _write_g(h * nout[...], g_o)


_ff_call = pl.pallas_call(
    _ff_body,
    out_shape=(
        jax.ShapeDtypeStruct((N, H), _f32),
        jax.ShapeDtypeStruct((2 * NPAD, HH), _f32),
    ),
)


def _head_body(h, p1W, p1b, p2W, p2b, l1W, l1b, l2W, l2b, pose_o, label_o):
    hh = h[...]
    z = jnp.maximum(jnp.dot(hh, p1W[...], preferred_element_type=_f32) + p1b[...], 0.0)
    pose_o[...] = jnp.dot(z, p2W[...], preferred_element_type=_f32) + p2b[...]
    y = jnp.mean(hh, axis=0, keepdims=True)
    u = jnp.maximum(jnp.dot(y, l1W[...], preferred_element_type=_f32) + l1b[...], 0.0)
    label_o[...] = jnp.dot(u, l2W[...], preferred_element_type=_f32) + l2b[...]


_head_call = pl.pallas_call(
    _head_body,
    out_shape=(
        jax.ShapeDtypeStruct((N, 3), _f32),
        jax.ShapeDtypeStruct((1, 60), _f32),
    ),
)


def kernel(edge_index, node_features, lap_pe, in_W, in_b, pe_W, pe_b,
           conv1_W, conv1_b, bn1_g, bn1_b, conv2_W, conv2_b, bn2_g, bn2_b,
           ff_W, ff_b, p1_W, p1_b, p2_W, p2_b, l1_W, l1_b, l2_W, l2_b):
    src = edge_index[0].reshape(NSUB, EPT)
    dst = edge_index[1].reshape(NSUB, EPT)
    padc = jnp.full((NSUB, NPAD - EPT), N, jnp.int32)
    src3 = jnp.concatenate([src, padc], axis=1).reshape(NSUB, NCH, CH)
    dst3 = jnp.concatenate([dst, padc], axis=1).reshape(NSUB, NCH, CH)
    degidx = jnp.concatenate([src3, dst3], axis=0)
    dst3p = dst3.reshape(2 * NSUB, NCH // 2, CH)
    sidx2 = jnp.concatenate([src3, src3 + NPAD], axis=0)

    deg = _deg_kernel(degidx)
    dout = deg[0:NPAD].reshape(NPAD, 1)
    din = deg[NPAD:2 * NPAD].reshape(NPAD, 1)

    r = lambda v: v.reshape(1, -1)
    h, g, nin, nout = _init_call(
        node_features, lap_pe, in_W, r(in_b), pe_W, r(pe_b), dout, din)

    for i in range(conv1_W.shape[0]):
        a = _mp_kernel(sidx2, dst3p, g)
        g = _mid_call(a, nin, nout, conv1_W[i], r(conv1_b[i]),
                      r(bn1_g[i]), r(bn1_b[i]))
        a = _mp_kernel(sidx2, dst3p, g)
        x = _x2_call(a, nin, conv2_W[i], r(conv2_b[i]),
                     r(bn2_g[i]), r(bn2_b[i]))
        h, g = _ff_call(x, ff_W[i], r(ff_b[i]), h, nout)

    pose, label = _head_call(h, p1_W, r(p1_b), p2_W, r(p2_b),
                             l1_W, r(l1_b), l2_W, r(l2_b))
    return (pose, label)


# speedup vs baseline: 3.4677x; 1.0033x over previous
import functools

import jax
import jax.numpy as jnp
from jax import lax
from jax.experimental import pallas as pl
from jax.experimental.pallas import tpu as pltpu
from jax.experimental.pallas import tpu_sc as plsc

N = 10000
NPAD = 10240
E = 160000
H = 256
HH = 128
NSUB = 16
EPT = E // NSUB
CH = 128
NCH = NPAD // CH
ROWS_PT = NPAD // NSUB

_MESH = plsc.VectorSubcoreMesh(core_axis_name="c", subcore_axis_name="s")
_f32 = jnp.float32


@functools.partial(
    pl.kernel,
    out_type=jax.ShapeDtypeStruct((2 * NPAD,), _f32),
    mesh=_MESH,
    scratch_types=[
        pltpu.VMEM((NCH, CH), jnp.int32),
        pltpu.VMEM((CH,), _f32),
        pltpu.VMEM((ROWS_PT,), _f32),
        pltpu.VMEM_SHARED((NPAD,), _f32),
    ],
)
def _deg_kernel(idx_hbm, deg_hbm, idx_v, ones_v, zero_v, acc_sh):
    c = lax.axis_index("c")
    s = lax.axis_index("s")

    @pl.loop(0, CH // 16)
    def _(i):
        ones_v[pl.ds(i * 16, 16)] = jnp.ones((16,), _f32)

    @pl.loop(0, ROWS_PT // 16)
    def _(i):
        zero_v[pl.ds(i * 16, 16)] = jnp.zeros((16,), _f32)

    pltpu.sync_copy(zero_v, acc_sh.at[pl.ds(s * ROWS_PT, ROWS_PT)])
    pltpu.sync_copy(idx_hbm.at[c * NSUB + s], idx_v)
    plsc.subcore_barrier()

    @pl.loop(0, NCH)
    def _(k):
        pltpu.sync_copy(ones_v, acc_sh.at[idx_v.at[k]], add=True)

    plsc.subcore_barrier()
    pltpu.sync_copy(acc_sh.at[pl.ds(s * ROWS_PT, ROWS_PT)],
                    deg_hbm.at[pl.ds(c * NPAD + s * ROWS_PT, ROWS_PT)])


@functools.partial(
    pl.kernel,
    out_type=jax.ShapeDtypeStruct((2 * NPAD, HH), _f32),
    mesh=_MESH,
    scratch_types=[
        pltpu.VMEM((NCH, CH), jnp.int32),
        pltpu.VMEM((NCH // 2, CH), jnp.int32),
        pltpu.VMEM((2, CH, HH), _f32),
        pltpu.VMEM_SHARED((NPAD, HH), _f32),
        pltpu.SemaphoreType.DMA,
        pltpu.SemaphoreType.DMA,
        pltpu.SemaphoreType.DMA,
        pltpu.SemaphoreType.DMA,
    ],
)
def _mp_kernel(sidx_hbm, didx_hbm, g_hbm, a_hbm, sidx, didx, rows, acc,
               gsem0, gsem1, ssem0, ssem1):
    c = lax.axis_index("c")
    s = lax.axis_index("s")

    @pl.loop(0, CH)
    def _(i):
        @pl.loop(0, HH // 16)
        def _(j):
            rows[0, i, pl.ds(j * 16, 16)] = jnp.zeros((16,), _f32)

    for k in range(ROWS_PT // CH):
        pltpu.async_copy(rows.at[0], acc.at[pl.ds(s * ROWS_PT + k * CH, CH)],
                         gsem0)
    for k in range(ROWS_PT // CH):
        pltpu.make_async_copy(rows.at[0],
                              acc.at[pl.ds(s * ROWS_PT, CH)], gsem0).wait()

    pltpu.sync_copy(sidx_hbm.at[c * NSUB + s], sidx)
    plsc.subcore_barrier()

    gsems = (gsem0, gsem1)
    ssems = (ssem0, ssem1)
    HCH = NCH // 2

    def _gather_start(k, b):
        pltpu.async_copy(g_hbm.at[sidx.at[k]], rows.at[b], gsems[b])

    def _gather_wait(k, b):
        pltpu.make_async_copy(g_hbm.at[sidx.at[k]], rows.at[b],
                              gsems[b]).wait()

    def _scatter_wait(b):
        pltpu.make_async_copy(rows.at[b], acc.at[didx.at[0]],
                              ssems[b]).wait()

    _gather_start(0, 0)

    for p in range(2):
        if p == 1:
            _scatter_wait((HCH - 1) % 2)
        pltpu.sync_copy(didx_hbm.at[s * 2 + p], didx)

        @pl.loop(p * HCH, (p + 1) * HCH, step=2)
        def _(k):
            for b in range(2):
                kk = k + b
                _gather_wait(kk, b)
                pltpu.async_copy(rows.at[b], acc.at[didx.at[kk - p * HCH]],
                                 ssems[b], add=True)

                @pl.when(kk + 1 < NCH)
                def _():
                    @pl.when(kk != p * HCH)
                    def _():
                        _scatter_wait(1 - b)

                    _gather_start(kk + 1, 1 - b)

    _scatter_wait(0)
    _scatter_wait(1)
    plsc.subcore_barrier()
    pltpu.sync_copy(acc.at[pl.ds(s * ROWS_PT, ROWS_PT)],
                    a_hbm.at[pl.ds(c * NPAD + s * ROWS_PT, ROWS_PT)])


def _write_g(g, g_o):
    pad = jnp.zeros((NPAD - N, HH), _f32)
    g_o[0:N, :] = g[:, 0:HH]
    g_o[N:NPAD, :] = pad
    g_o[NPAD:NPAD + N, :] = g[:, HH:H]
    g_o[NPAD + N:2 * NPAD, :] = pad


def _init_body(nf, pe, inW, inb, peW, peb, dout, din,
               h_o, g_o, nin_o, nout_o):
    do = dout[0:N, :]
    di = din[0:N, :]
    nout = jnp.where(do > 0, 1.0 / jnp.sqrt(do), 0.0)
    nin = jnp.where(di > 0, 1.0 / jnp.sqrt(di), 0.0)
    h = (jnp.dot(nf[...], inW[...], preferred_element_type=_f32) + inb[...]
         + jnp.dot(pe[...], peW[...], preferred_element_type=_f32) + peb[...])
    h_o[...] = h
    nin_o[...] = nin
    nout_o[...] = nout
    _write_g(h * nout, g_o)


_init_call = pl.pallas_call(
    _init_body,
    out_shape=(
        jax.ShapeDtypeStruct((N, H), _f32),
        jax.ShapeDtypeStruct((2 * NPAD, HH), _f32),
        jax.ShapeDtypeStruct((N, 1), _f32),
        jax.ShapeDtypeStruct((N, 1), _f32),
    ),
)


def _gconv_post(a, nin, W, b, bng, bnb):
    af = jnp.concatenate([a[0:N, :], a[NPAD:NPAD + N, :]], axis=1) * nin
    t = jnp.dot(af, W[...], preferred_element_type=_f32) + b
    mu = jnp.mean(t, axis=0, keepdims=True)
    d = t - mu
    var = jnp.mean(d * d, axis=0, keepdims=True)
    return jnp.maximum(d * (1.0 / jnp.sqrt(var + 1e-5)) * bng + bnb, 0.0)


def _mid_body(a, nin, nout, W, b, bng, bnb, g_o):
    x = _gconv_post(a[...], nin[...], W[...], b[...], bng[...], bnb[...])
    _write_g(x * nout[...], g_o)


_mid_call = pl.pallas_call(
    _mid_body,
    out_shape=jax.ShapeDtypeStruct((2 * NPAD, HH), _f32),
)


def _x2_body(a, nin, W, b, bng, bnb, x_o):
    x_o[...] = _gconv_post(a[...], nin[...], W[...], b[...], bng[...],
                           bnb[...])


_x2_call = pl.pallas_call(
    _x2_body,
    out_shape=jax.ShapeDtypeStruct((N, H), _f32),
)


def _ff_body(x, ffW, ffb, hin, nout, h_o, g_o):
    h = jnp.dot(x[...], ffW[...], preferred_element_type=_f32) + ffb[...] + hin[...]
    h_o[...] = h
    _write_g(h * nout[...], g_o)


_ff_call = pl.pallas_call(
    _ff_body,
    out_shape=(
        jax.ShapeDtypeStruct((N, H), _f32),
        jax.ShapeDtypeStruct((2 * NPAD, HH), _f32),
    ),
)


def _head_body(h, p1W, p1b, p2W, p2b, l1W, l1b, l2W, l2b, pose_o, label_o):
    hh = h[...]
    z = jnp.maximum(jnp.dot(hh, p1W[...], preferred_element_type=_f32) + p1b[...], 0.0)
    pose_o[...] = jnp.dot(z, p2W[...], preferred_element_type=_f32) + p2b[...]
    y = jnp.mean(hh, axis=0, keepdims=True)
    u = jnp.maximum(jnp.dot(y, l1W[...], preferred_element_type=_f32) + l1b[...], 0.0)
    label_o[...] = jnp.dot(u, l2W[...], preferred_element_type=_f32) + l2b[...]


_head_call = pl.pallas_call(
    _head_body,
    out_shape=(
        jax.ShapeDtypeStruct((N, 3), _f32),
        jax.ShapeDtypeStruct((1, 60), _f32),
    ),
)


def kernel(edge_index, node_features, lap_pe, in_W, in_b, pe_W, pe_b,
           conv1_W, conv1_b, bn1_g, bn1_b, conv2_W, conv2_b, bn2_g, bn2_b,
           ff_W, ff_b, p1_W, p1_b, p2_W, p2_b, l1_W, l1_b, l2_W, l2_b):
    src = edge_index[0].reshape(NSUB, EPT)
    dst = edge_index[1].reshape(NSUB, EPT)
    padc = jnp.full((NSUB, NPAD - EPT), N, jnp.int32)
    src3 = jnp.concatenate([src, padc], axis=1).reshape(NSUB, NCH, CH)
    dst3 = jnp.concatenate([dst, padc], axis=1).reshape(NSUB, NCH, CH)
    degidx = jnp.concatenate([src3, dst3], axis=0)
    dst3p = dst3.reshape(2 * NSUB, NCH // 2, CH)
    sidx2 = jnp.concatenate([src3, src3 + NPAD], axis=0)

    deg = _deg_kernel(degidx)
    dout = deg[0:NPAD].reshape(NPAD, 1)
    din = deg[NPAD:2 * NPAD].reshape(NPAD, 1)

    r = lambda v: v.reshape(1, -1)
    h, g, nin, nout = _init_call(
        node_features, lap_pe, in_W, r(in_b), pe_W, r(pe_b), dout, din)

    for i in range(conv1_W.shape[0]):
        a = _mp_kernel(sidx2, dst3p, g)
        g = _mid_call(a, nin, nout, conv1_W[i], r(conv1_b[i]),
                      r(bn1_g[i]), r(bn1_b[i]))
        a = _mp_kernel(sidx2, dst3p, g)
        x = _x2_call(a, nin, conv2_W[i], r(conv2_b[i]),
                     r(bn2_g[i]), r(bn2_b[i]))
        h, g = _ff_call(x, ff_W[i], r(ff_b[i]), h, nout)

    pose, label = _head_call(h, p1_W, r(p1_b), p2_W, r(p2_b),
                             l1_W, r(l1_b), l2_W, r(l2_b))
    return (pose, label)
